# baseline (speedup 1.0000x reference)
import functools

import jax
import jax.numpy as jnp
from jax import lax
from jax.experimental import pallas as pl
from jax.experimental.pallas import tpu as pltpu
from jax.experimental.pallas import tpu_sc as plsc

_NC = 2
_NS = 16
_NW = _NC * _NS
_L = 16
_TPAD = 56
_IPAD = 64


def _lse_body(t_ref, o_ref):
    t = t_ref[...]
    m = jnp.max(t, axis=1, keepdims=True)
    o_ref[...] = m + jnp.log(jnp.sum(jnp.exp(t - m), axis=1, keepdims=True))


def _finalize_body(inv_n, p_ref, o_ref):
    o_ref[...] = jnp.sum(p_ref[...]) * inv_n * jnp.ones((1, 1), jnp.float32)


def _sc_body(xp_hbm, yp_hbm, tab_hbm, lse_hbm, out_hbm, part_hbm,
             idx_v, y_v, rows_v, lse_v, acc_v, red_v, shared, sem):
    c = lax.axis_index("c")
    s = lax.axis_index("s")
    wid = s * _NC + c
    nb = out_hbm.shape[0] // _NW
    ntok = out_hbm.shape[1]

    pltpu.sync_copy(lse_hbm, lse_v)
    acc_v[...] = jnp.zeros((_L,), jnp.float32)
    for q in range(_IPAD // _L):
        idx_v[pl.ds(q * _L, _L)] = jnp.zeros((_L,), jnp.int32)
        y_v[pl.ds(q * _L, _L)] = jnp.zeros((_L,), jnp.int32)

    def chunk_body(k, carry):
        b = wid * nb + k
        pltpu.sync_copy(xp_hbm.at[b], idx_v.at[pl.ds(0, _TPAD)])
        pltpu.sync_copy(yp_hbm.at[b], y_v.at[pl.ds(0, _TPAD)])
        pltpu.async_copy(tab_hbm.at[idx_v], rows_v, sem).wait()
        pltpu.sync_copy(rows_v.at[pl.ds(0, ntok)], out_hbm.at[b])
        for g in range(_IPAD // _L):
            pos = lax.iota(jnp.int32, _L) + (g * _L)
            xg = idx_v[pl.ds(g * _L, _L)]
            yg = y_v[pl.ds(g * _L, _L)]
            lvals = plsc.load_gather(lse_v, [xg])
            tvals = plsc.load_gather(rows_v, [pos, yg])
            acc_v[...] = acc_v[...] + jnp.where(pos < ntok, lvals - tvals, 0.0)
        return carry

    lax.fori_loop(0, nb, chunk_body, 0)

    pltpu.sync_copy(acc_v, shared.at[s])
    plsc.subcore_barrier()

    @pl.when(s == 0)
    def _():
        pltpu.sync_copy(shared, red_v)
        tot = jnp.zeros((_L,), jnp.float32)
        for i in range(_NS):
            tot = tot + red_v[i, :]
        acc_v[...] = tot
        pltpu.sync_copy(acc_v, part_hbm.at[c])


def _make_sc_call(bsz, ntok, v, d):
    mesh = plsc.VectorSubcoreMesh(
        core_axis_name="c", subcore_axis_name="s",
        num_cores=_NC, num_subcores=_NS)
    return pl.kernel(
        _sc_body,
        out_type=[
            jax.ShapeDtypeStruct((bsz, ntok, d), jnp.float32),
            jax.ShapeDtypeStruct((_NC, _L), jnp.float32),
        ],
        mesh=mesh,
        compiler_params=pltpu.CompilerParams(
            needs_layout_passes=False, use_tc_tiling_on_sc=False),
        scratch_types=[
            pltpu.VMEM((_IPAD,), jnp.int32),
            pltpu.VMEM((_IPAD,), jnp.int32),
            pltpu.VMEM((_IPAD, d), jnp.float32),
            pltpu.VMEM((1024,), jnp.float32),
            pltpu.VMEM((_L,), jnp.float32),
            pltpu.VMEM((_NS, _L), jnp.float32),
            pltpu.VMEM_SHARED((_NS, _L), jnp.float32),
            pltpu.SemaphoreType.DMA,
        ],
    )


def kernel(x, y, table):
    b, t = x.shape
    v, d = table.shape
    n = b * t
    xp = jnp.pad(x.astype(jnp.int32), ((0, 0), (0, _TPAD - t)))
    yp = jnp.pad(y.astype(jnp.int32), ((0, 0), (0, _TPAD - t)))

    lse = pl.pallas_call(
        _lse_body,
        out_shape=jax.ShapeDtypeStruct((v, 1), jnp.float32),
    )(table)
    lse_pad = jnp.pad(lse.reshape(v), (0, 1024 - v))

    logits, parts = _make_sc_call(b, t, v, d)(xp, yp, table, lse_pad)

    loss = pl.pallas_call(
        functools.partial(_finalize_body, 1.0 / n),
        out_shape=jax.ShapeDtypeStruct((1, 1), jnp.float32),
    )(parts)

    return logits, loss[0, 0]


# speedup vs baseline: 2.2285x; 2.2285x over previous
import functools

import jax
import jax.numpy as jnp
from jax import lax
from jax.experimental import pallas as pl
from jax.experimental.pallas import tpu as pltpu
from jax.experimental.pallas import tpu_sc as plsc

_NC = 2
_NS = 16
_NW = _NC * _NS
_L = 16
_CHUNK = 64
_NP = 8
_BB = 4


def _lse_body(t_ref, o_ref):
    t = t_ref[...]
    m = jnp.max(t, axis=1, keepdims=True)
    o_ref[...] = m + jnp.log(jnp.sum(jnp.exp(t - m), axis=1, keepdims=True))


def _finalize_body(inv_n, p_ref, o_ref):
    o_ref[...] = jnp.sum(p_ref[...]) * inv_n * jnp.ones((1, 1), jnp.float32)


def _fmt_body(ntok, d, in_ref, o_ref):
    x = in_ref[...]
    outs = []
    for bb in range(_BB):
        pieces = [x[p, bb * ntok:(bb + 1) * ntok, :] for p in range(_NP)]
        outs.append(jnp.concatenate(pieces, axis=-1)[:, :d])
    o_ref[...] = jnp.stack(outs, axis=0)


def _sc_body(x_hbm, y_hbm, t2_hbm, lse_hbm, out8_hbm, part_hbm,
             idx_v, y_v, rows_v, lse_v, acc_v, gsem):
    c = lax.axis_index("c")
    s = lax.axis_index("s")
    wid = s * _NC + c
    n = x_hbm.shape[0]
    per_w = n // _NW
    nchunk = per_w // _CHUNK

    pltpu.sync_copy(lse_hbm, lse_v)
    acc_v[...] = jnp.zeros((_L,), jnp.float32)

    def chunk_body(k, carry):
        base = wid * per_w + k * _CHUNK
        pltpu.sync_copy(x_hbm.at[pl.ds(base, _CHUNK)], idx_v)
        pltpu.sync_copy(y_hbm.at[pl.ds(base, _CHUNK)], y_v)
        pltpu.async_copy(t2_hbm.at[idx_v], rows_v, gsem).wait()
        for p in range(_NP):
            pltpu.sync_copy(rows_v.at[:, pl.ds(p * 128, 128)],
                            out8_hbm.at[p, pl.ds(base, _CHUNK)])
        for g in range(_CHUNK // _L):
            rid = lax.iota(jnp.int32, _L) + (g * _L)
            xg = idx_v[pl.ds(g * _L, _L)]
            yg = y_v[pl.ds(g * _L, _L)]
            lvals = plsc.load_gather(lse_v, [xg])
            tvals = plsc.load_gather(rows_v, [rid, yg])
            acc_v[...] = acc_v[...] + (lvals - tvals)
        return carry

    lax.fori_loop(0, nchunk, chunk_body, 0)

    pltpu.sync_copy(acc_v, part_hbm.at[wid])


def _make_sc_call(n):
    mesh = plsc.VectorSubcoreMesh(
        core_axis_name="c", subcore_axis_name="s",
        num_cores=_NC, num_subcores=_NS)
    return pl.kernel(
        _sc_body,
        out_type=[
            jax.ShapeDtypeStruct((_NP, n, 128), jnp.float32),
            jax.ShapeDtypeStruct((_NW, _L), jnp.float32),
        ],
        mesh=mesh,
        compiler_params=pltpu.CompilerParams(needs_layout_passes=False),
        scratch_types=[
            pltpu.VMEM((_CHUNK,), jnp.int32),
            pltpu.VMEM((_CHUNK,), jnp.int32),
            pltpu.VMEM((_CHUNK, 1024), jnp.float32),
            pltpu.VMEM((1024,), jnp.float32),
            pltpu.VMEM((_L,), jnp.float32),
            pltpu.SemaphoreType.DMA,
        ],
    )


def kernel(x, y, table):
    b, t = x.shape
    v, d = table.shape
    n = b * t
    xf = x.reshape(n).astype(jnp.int32)
    yf = y.reshape(n).astype(jnp.int32)
    t2 = jnp.pad(table, ((0, 0), (0, _NP * 128 - d)))

    lse = pl.pallas_call(
        _lse_body,
        out_shape=jax.ShapeDtypeStruct((v, 1), jnp.float32),
    )(table)
    lse_pad = jnp.pad(lse.reshape(v), (0, 1024 - v))

    out8, parts = _make_sc_call(n)(xf, yf, t2, lse_pad)

    logits = pl.pallas_call(
        functools.partial(_fmt_body, t, d),
        grid=(b // _BB,),
        in_specs=[pl.BlockSpec((_NP, _BB * t, 128), lambda g: (0, g, 0))],
        out_specs=pl.BlockSpec((_BB, t, d), lambda g: (g, 0, 0)),
        out_shape=jax.ShapeDtypeStruct((b, t, d), jnp.float32),
    )(out8)

    loss = pl.pallas_call(
        functools.partial(_finalize_body, 1.0 / n),
        out_shape=jax.ShapeDtypeStruct((1, 1), jnp.float32),
    )(parts)

    return logits, loss[0, 0]
